# Initial kernel scaffold; baseline (speedup 1.0000x reference)
import jax
import jax.numpy as jnp
from jax.experimental import pallas as pl


def kernel(x, params):
    raise NotImplementedError("write your pallas kernel here")


# speedup vs baseline: 6.3519x; 6.3519x over previous
import functools
import numpy as np
import jax
import jax.numpy as jnp
from jax import lax
from jax.experimental import pallas as pl
from jax.experimental.pallas import tpu as pltpu
from jax.experimental.pallas import tpu_sc as plsc

B, S, D, H, HD = 8, 2048, 256, 4, 64
DFF = 4 * D
NE = 4
SCALE = 1.0 / float(np.sqrt(HD))
EPS = 1e-6

SC_NC, SC_NS = 2, 16
SC_NW = SC_NC * SC_NS


def _np_rope_tables():
    theta = 1.0 / (10000.0 ** (np.arange(0, HD, 2).astype(np.float32) / HD))
    fr = np.outer(np.arange(S).astype(np.float32), theta)
    cf = np.repeat(np.cos(fr), 2, axis=1)
    sf = np.repeat(np.sin(fr), 2, axis=1)
    return (np.tile(cf, (1, H)).astype(np.float32),
            np.tile(sf, (1, H)).astype(np.float32))


def _np_rot_mat():
    r = np.arange(D)[:, None]
    c = np.arange(D)[None, :]
    P = (np.where((r == c + 1) & (c % 2 == 0), -1.0, 0.0)
         + np.where((r == c - 1) & (c % 2 == 1), 1.0, 0.0))
    return P.astype(np.float32)


_CF, _SF = _np_rope_tables()
_PM = _np_rot_mat()
_LTGE = np.tril(np.ones((S, S), np.float32))
_IEQ = np.eye(S, dtype=np.float32)


def _rms(x):
    return x * lax.rsqrt(jnp.mean(x * x, axis=-1, keepdims=True) + EPS)


def _dot(a, b):
    return jnp.dot(a, b, preferred_element_type=jnp.float32)


def _attn_body(mode, *refs):
    if mode == 'dec':
        x_ref, add_ref, wq_ref, wkv_ref, cf_ref, sf_ref, p_ref, o_ref = refs
        x = x_ref[0] + add_ref[0]
    elif mode == 'main':
        x_ref, rm_ref, wq_ref, wkv_ref, cf_ref, sf_ref, p_ref, o_ref = refs
        x = x_ref[0] * rm_ref[0]
    else:
        x_ref, wq_ref, wkv_ref, cf_ref, sf_ref, p_ref, o_ref = refs
        x = x_ref[0]
    xn = _rms(x)
    q = _dot(xn, wq_ref[...])
    q = q * cf_ref[...] + _dot(q, p_ref[...]) * sf_ref[...]
    kv = _dot(xn, wkv_ref[...])
    ks = [kv[:, j * 2 * HD: j * 2 * HD + HD] for j in range(H)]
    vs = [kv[:, j * 2 * HD + HD: (j + 1) * 2 * HD] for j in range(H)]
    for i in range(H):
        qi = q[:, i * HD:(i + 1) * HD]
        lg = [jnp.sum(qi * ks[j], axis=1, keepdims=True) * SCALE
              for j in range(H)]
        m = lg[0]
        for j in range(1, H):
            m = jnp.maximum(m, lg[j])
        es = [jnp.exp(l - m) for l in lg]
        den = es[0] + es[1] + es[2] + es[3]
        acc = es[0] * vs[0]
        for j in range(1, H):
            acc = acc + es[j] * vs[j]
        ao = acc / den
        if mode == 'main':
            o_ref[0, i] = jnp.concatenate([ao, ao], axis=1)
        else:
            o_ref[0, i] = ao


def _attn(mode, x, extra, wqT, wkvT):
    full = lambda a: pl.BlockSpec(a.shape, lambda b: (0,) * a.ndim)
    specs = [pl.BlockSpec((1, S, D), lambda b: (b, 0, 0))]
    args = [x]
    if mode == 'dec':
        specs.append(pl.BlockSpec((1, S, D), lambda b: (b, 0, 0)))
        args.append(extra)
    elif mode == 'main':
        specs.append(pl.BlockSpec((1, S, 1), lambda b: (b, 0, 0)))
        args.append(extra)
    consts = [wqT, wkvT, jnp.asarray(_CF), jnp.asarray(_SF), jnp.asarray(_PM)]
    specs += [full(a) for a in consts]
    args += consts
    od = 2 * HD if mode == 'main' else HD
    return pl.pallas_call(
        functools.partial(_attn_body, mode),
        grid=(B,),
        in_specs=specs,
        out_specs=pl.BlockSpec((1, H, S, od), lambda b: (b, 0, 0, 0)),
        out_shape=jax.ShapeDtypeStruct((B, H, S, od), jnp.float32),
    )(*args)


def _proj_body(mode, *refs):
    if mode == 'dec':
        a_ref, i1_ref, i2_ref, wo_ref, o_ref = refs
        ident = i1_ref[0] + i2_ref[0]
    elif mode == 'main':
        a_ref, i1_ref, rm_ref, wo_ref, o_ref = refs
        ident = i1_ref[0] * rm_ref[0]
    else:
        a_ref, i1_ref, wo_ref, o_ref = refs
        ident = i1_ref[0]
    o_ref[0] = _dot(a_ref[0], wo_ref[...]) + ident


def _proj(mode, a, idents, woT):
    xspec = pl.BlockSpec((1, S, D), lambda b: (b, 0, 0))
    rmspec = pl.BlockSpec((1, S, 1), lambda b: (b, 0, 0))
    if mode == 'dec':
        specs = [xspec, xspec, xspec]
    elif mode == 'main':
        specs = [xspec, xspec, rmspec]
    else:
        specs = [xspec, xspec]
    args = [a] + list(idents)
    specs.append(pl.BlockSpec(woT.shape, lambda b: (0, 0)))
    args.append(woT)
    return pl.pallas_call(
        functools.partial(_proj_body, mode),
        grid=(B,),
        in_specs=specs,
        out_specs=xspec,
        out_shape=jax.ShapeDtypeStruct((B, S, D), jnp.float32),
    )(*args)


def _moe_body(y_ref, w1_ref, w3_ref, w2_ref, g_ref, o_ref):
    y = y_ref[0]
    yn = _rms(y)
    gl = _dot(yn, g_ref[...])
    m = jnp.max(gl, axis=1, keepdims=True)
    eg = jnp.exp(gl - m)
    gw = eg / jnp.sum(eg, axis=1, keepdims=True)
    cols = [gw[:, i:i + 1] for i in range(NE)]
    sel = []
    for i in range(NE):
        cnt = None
        for j in range(NE):
            if j == i:
                continue
            c = (cols[j] > cols[i]).astype(jnp.float32)
            cnt = c if cnt is None else cnt + c
        sel.append(cols[i] * (cnt < 2.0).astype(jnp.float32))
    wsum = sel[0] + sel[1] + sel[2] + sel[3]
    out = y
    for i in range(NE + 1):
        a1 = _dot(yn, w1_ref[i])
        a3 = _dot(yn, w3_ref[i])
        gel = 0.5 * a1 * (1.0 + lax.erf(a1 * 0.7071067811865476))
        eo = _dot(gel * a3, w2_ref[i])
        if i < NE:
            out = out + eo * (sel[i] / wsum)
        else:
            out = out + eo
    o_ref[0] = out


def _moe(y, w1T, w3T, w2T, gT):
    full = lambda a: pl.BlockSpec(a.shape, lambda b: (0,) * a.ndim)
    xspec = pl.BlockSpec((1, S, D), lambda b: (b, 0, 0))
    return pl.pallas_call(
        _moe_body,
        grid=(B,),
        in_specs=[xspec, full(w1T), full(w3T), full(w2T), full(gT)],
        out_specs=xspec,
        out_shape=jax.ShapeDtypeStruct((B, S, D), jnp.float32),
    )(y, w1T, w3T, w2T, gT)


def _routing_body(m_ref, ltge_ref, ieq_ref,
                  i1_ref, i2_ref, i3_ref, rm_ref):
    ones_row = jnp.ones((1, S), jnp.float32)
    iota_lane = lax.broadcasted_iota(jnp.int32, (S, S), 1).astype(jnp.float32)
    iota1s = lax.broadcasted_iota(jnp.int32, (1, S), 1).astype(jnp.float32)
    counts = []
    for b in range(B):
        maskcol = m_ref[b]
        cum_col = jnp.dot(ltge_ref[...], maskcol,
                          preferred_element_type=jnp.float32)
        cnt = jnp.max(cum_col)
        counts.append(cnt)
        cum_row = jnp.dot(ones_row, ieq_ref[...] * cum_col,
                          preferred_element_type=jnp.float32,
                          precision=lax.Precision.HIGHEST)
        i3_ref[b] = (cum_row.astype(jnp.int32) - 1 + b * S)
        cmat = (cum_col <= iota_lane).astype(jnp.float32)
        pos_row = jnp.minimum(
            jnp.dot(ones_row, cmat, preferred_element_type=jnp.float32),
            float(S - 1))
        i1_ref[b] = (pos_row.astype(jnp.int32) + b * S)
        rm_ref[b] = (iota1s < cnt).astype(jnp.float32)
    seq = counts[0]
    for b in range(1, B):
        seq = jnp.maximum(seq, counts[b])
    seq_i = seq.astype(jnp.int32)
    g = lax.broadcasted_iota(jnp.int32, (1, 4 * S), 1)
    hx = ((g >= seq_i).astype(jnp.int32)
          + (g >= 2 * seq_i).astype(jnp.int32)
          + (g >= 3 * seq_i).astype(jnp.int32))
    rem = jnp.minimum(g - hx * seq_i, S - 1)
    loc = hx * S + rem
    for b in range(B):
        i2_ref[b] = loc + b * 4 * S


def _routing(mask3):
    full = lambda a: pl.BlockSpec(a.shape, lambda: (0,) * a.ndim)
    ltge = jnp.asarray(_LTGE)
    ieq = jnp.asarray(_IEQ)
    return pl.pallas_call(
        _routing_body,
        in_specs=[full(mask3), full(ltge), full(ieq)],
        out_specs=(
            pl.BlockSpec((B, 1, S), lambda: (0, 0, 0)),
            pl.BlockSpec((B, 1, 4 * S), lambda: (0, 0, 0)),
            pl.BlockSpec((B, 1, S), lambda: (0, 0, 0)),
            pl.BlockSpec((B, 1, S), lambda: (0, 0, 0)),
        ),
        out_shape=(
            jax.ShapeDtypeStruct((B, 1, S), jnp.int32),
            jax.ShapeDtypeStruct((B, 1, 4 * S), jnp.int32),
            jax.ShapeDtypeStruct((B, 1, S), jnp.int32),
            jax.ShapeDtypeStruct((B, 1, S), jnp.float32),
        ),
    )(mask3, ltge, ieq)


def _sc_gather(src, idx, chunk):
    M = idx.shape[0]
    Dm = src.shape[1]
    per_w = M // SC_NW
    rounds = per_w // chunk
    assert per_w % chunk == 0 and M % SC_NW == 0
    mesh = plsc.VectorSubcoreMesh(core_axis_name="c", subcore_axis_name="s")

    @functools.partial(
        pl.kernel, mesh=mesh,
        out_type=jax.ShapeDtypeStruct((M, Dm), jnp.float32),
        scratch_types=[
            pltpu.VMEM((chunk,), jnp.int32),
            pltpu.VMEM((chunk, Dm), jnp.float32),
            pltpu.SemaphoreType.DMA,
        ],
    )
    def k(src_hbm, idx_hbm, out_hbm, idx_v, rows_v, sem):
        wid = lax.axis_index("s") * SC_NC + lax.axis_index("c")
        for r in range(rounds):
            base = wid * per_w + r * chunk
            pltpu.sync_copy(idx_hbm.at[pl.ds(base, chunk)], idx_v)
            pltpu.async_copy(src_hbm.at[idx_v], rows_v, sem).wait()
            pltpu.sync_copy(rows_v, out_hbm.at[pl.ds(base, chunk)])

    return k(src, idx)


def _r_rmsnorm(x, eps=1e-06):
    return x / jnp.sqrt(jnp.mean(x * x, axis=-1, keepdims=True) + eps)


def _r_freqs(seq_len, head_dim):
    theta = 1.0 / (10000.0 ** (np.arange(0, head_dim, 2).astype(np.float32)
                               / head_dim))
    pos = np.arange(seq_len).astype(np.float32)
    freqs = np.outer(pos, theta)
    return jnp.asarray(np.exp(1j * freqs).astype(np.complex64))


def _r_rotary(x, freqs_cis):
    Bb, Ss, Hh, hd = x.shape
    xc = x.reshape(Bb, Ss, Hh, hd // 2, 2)
    xcplx = jax.lax.complex(xc[..., 0], xc[..., 1])
    f = freqs_cis[:Ss].reshape(1, Ss, 1, hd // 2)
    rot = xcplx * f
    return jnp.stack([jnp.real(rot), jnp.imag(rot)], axis=-1).reshape(
        Bb, Ss, Hh, hd)


def _r_mla(x, p, freqs_cis):
    Bb, Ss, Dd = x.shape
    identity = x
    xn = _r_rmsnorm(x)
    q = (xn @ p['wq'].T).reshape(Bb, Ss, H, HD)
    kv = (xn @ p['wkv'].T).reshape(Bb, Ss, H, 2 * HD)
    k = kv[..., :HD]
    v = kv[..., HD:]
    q = _r_rotary(q, freqs_cis)
    attn = jnp.matmul(q, jnp.swapaxes(k, -1, -2)) / np.sqrt(HD)
    attn = jax.nn.softmax(attn.astype(jnp.float32), axis=-1).astype(x.dtype)
    ao = jnp.matmul(attn, v)
    ao = jnp.transpose(ao, (0, 2, 1, 3)).reshape(Bb, Ss, Dd)
    return ao @ p['wo'].T + identity


def _r_expert_ff(x, ep):
    return (jax.nn.gelu(x @ ep['w1'].T, approximate=False)
            * (x @ ep['w3'].T)) @ ep['w2'].T


def _r_moe(x, p):
    identity = x
    xn = _r_rmsnorm(x)
    gw = jax.nn.softmax(xn @ p['gate'].T, axis=-1)
    topw, topi = jax.lax.top_k(gw, 2)
    topw = topw / jnp.sum(topw, axis=-1, keepdims=True)
    out = jnp.zeros_like(x)
    for i in range(NE):
        eo = _r_expert_ff(xn, p['experts'][i])
        wi = jnp.sum(topw * (topi == i).astype(x.dtype), axis=-1,
                     keepdims=True)
        out = out + eo * wi
    out = out + _r_expert_ff(xn, p['shared'])
    return out + identity


def _r_mask(x, params):
    freqs = _r_freqs(2048, HD)
    hh = _r_moe(_r_mla(x, params['encoder'], freqs), params['encoder'])
    pr = params['routing']
    q = hh[:, :-1] @ pr['wq'].T
    k = hh[:, 1:] @ pr['wk'].T
    eps = 1e-08
    qn = jnp.maximum(jnp.linalg.norm(q, axis=-1), eps)
    kn = jnp.maximum(jnp.linalg.norm(k, axis=-1), eps)
    cos = jnp.sum(q * k, axis=-1) / (qn * kn)
    p = jnp.clip((1.0 - cos) / 2.0, 0.0, 1.0)
    p = jnp.pad(p, ((0, 0), (1, 0)), constant_values=1.0)
    return (p > 0.5).astype(jnp.float32).reshape(B, S, 1)


def _block_std(x, p, add=None):
    wqT = p['wq'].T
    wkvT = p['wkv'].T
    woT = p['wo'].T
    w1T = jnp.stack([e['w1'].T for e in (p['experts'] + [p['shared']])])
    w3T = jnp.stack([e['w3'].T for e in (p['experts'] + [p['shared']])])
    w2T = jnp.stack([e['w2'].T for e in (p['experts'] + [p['shared']])])
    gT = p['gate'].T
    if add is None:
        ao = _attn('enc', x, None, wqT, wkvT)
        y = _proj('enc', ao.reshape(B, S, D), (x,), woT)
    else:
        ao = _attn('dec', x, add, wqT, wkvT)
        y = _proj('dec', ao.reshape(B, S, D), (x, add), woT)
    return _moe(y, w1T, w3T, w2T, gT)


def kernel(x, params):
    pe, pm, pd = params['encoder'], params['main'], params['decoder']
    pr = params['routing']

    h = _block_std(x, pe)
    mask3 = _r_mask(x, params)

    idx1, idx2, idx3, rm = _routing(mask3)
    idx1 = idx1.reshape(B * S)
    idx2 = idx2.reshape(B * 4 * S)
    idx3 = idx3.reshape(B * S)
    rm = rm.reshape(B, S, 1)

    comp = _sc_gather(h.reshape(B * S, D), idx1, 128).reshape(B, S, D)

    ao = _attn('main', comp, rm, pm['wq'].T, pm['wkv'].T)
    aop = _sc_gather(ao.reshape(B * H * S, 2 * HD), idx2, 128)
    aop = aop[:, :HD].reshape(B, S, D)
    y2 = _proj('main', aop, (comp, rm), pm['wo'].T)
    w1T = jnp.stack([e['w1'].T for e in (pm['experts'] + [pm['shared']])])
    w3T = jnp.stack([e['w3'].T for e in (pm['experts'] + [pm['shared']])])
    w2T = jnp.stack([e['w2'].T for e in (pm['experts'] + [pm['shared']])])
    c2 = _moe(y2, w1T, w3T, w2T, pm['gate'].T)

    h2 = _sc_gather(c2.reshape(B * S, D), idx3, 128).reshape(B, S, D)

    return _block_std(h, pd, add=h2)


# speedup vs baseline: 6.3670x; 1.0024x over previous
import functools
import numpy as np
import jax
import jax.numpy as jnp
from jax import lax
from jax.experimental import pallas as pl
from jax.experimental.pallas import tpu as pltpu
from jax.experimental.pallas import tpu_sc as plsc

B, S, D, H, HD = 8, 2048, 256, 4, 64
DFF = 4 * D
NE = 4
SCALE = 1.0 / float(np.sqrt(HD))
EPS = 1e-6

SC_NC, SC_NS = 2, 16
SC_NW = SC_NC * SC_NS


def _np_rope_tables():
    theta = 1.0 / (10000.0 ** (np.arange(0, HD, 2).astype(np.float32) / HD))
    fr = np.outer(np.arange(S).astype(np.float32), theta)
    cf = np.repeat(np.cos(fr), 2, axis=1)
    sf = np.repeat(np.sin(fr), 2, axis=1)
    return (np.tile(cf, (1, H)).astype(np.float32),
            np.tile(sf, (1, H)).astype(np.float32))


def _np_rot_mat():
    r = np.arange(D)[:, None]
    c = np.arange(D)[None, :]
    P = (np.where((r == c + 1) & (c % 2 == 0), -1.0, 0.0)
         + np.where((r == c - 1) & (c % 2 == 1), 1.0, 0.0))
    return P.astype(np.float32)


_CF, _SF = _np_rope_tables()
_PM = _np_rot_mat()
_LTGE = np.tril(np.ones((S, S), np.float32))
_IEQ = np.eye(S, dtype=np.float32)


def _rms(x):
    return x * lax.rsqrt(jnp.mean(x * x, axis=-1, keepdims=True) + EPS)


def _dot(a, b):
    return jnp.dot(a, b, preferred_element_type=jnp.float32)


def _attn_body(mode, *refs):
    if mode == 'dec':
        x_ref, add_ref, wq_ref, wkv_ref, cf_ref, sf_ref, p_ref, o_ref = refs
        x = x_ref[0] + add_ref[0]
    elif mode == 'main':
        x_ref, rm_ref, wq_ref, wkv_ref, cf_ref, sf_ref, p_ref, o_ref = refs
        x = x_ref[0] * rm_ref[0]
    else:
        x_ref, wq_ref, wkv_ref, cf_ref, sf_ref, p_ref, o_ref = refs
        x = x_ref[0]
    xn = _rms(x)
    q = _dot(xn, wq_ref[...])
    q = q * cf_ref[...] + _dot(q, p_ref[...]) * sf_ref[...]
    kv = _dot(xn, wkv_ref[...])
    ks = [kv[:, j * 2 * HD: j * 2 * HD + HD] for j in range(H)]
    vs = [kv[:, j * 2 * HD + HD: (j + 1) * 2 * HD] for j in range(H)]
    for i in range(H):
        qi = q[:, i * HD:(i + 1) * HD]
        lg = [jnp.sum(qi * ks[j], axis=1, keepdims=True) * SCALE
              for j in range(H)]
        m = lg[0]
        for j in range(1, H):
            m = jnp.maximum(m, lg[j])
        es = [jnp.exp(l - m) for l in lg]
        den = es[0] + es[1] + es[2] + es[3]
        acc = es[0] * vs[0]
        for j in range(1, H):
            acc = acc + es[j] * vs[j]
        ao = acc / den
        if mode == 'main':
            o_ref[0, i] = jnp.concatenate([ao, ao], axis=1)
        else:
            o_ref[0, i] = ao


def _attn(mode, x, extra, wqT, wkvT):
    full = lambda a: pl.BlockSpec(a.shape, lambda b: (0,) * a.ndim)
    specs = [pl.BlockSpec((1, S, D), lambda b: (b, 0, 0))]
    args = [x]
    if mode == 'dec':
        specs.append(pl.BlockSpec((1, S, D), lambda b: (b, 0, 0)))
        args.append(extra)
    elif mode == 'main':
        specs.append(pl.BlockSpec((1, S, 1), lambda b: (b, 0, 0)))
        args.append(extra)
    consts = [wqT, wkvT, jnp.asarray(_CF), jnp.asarray(_SF), jnp.asarray(_PM)]
    specs += [full(a) for a in consts]
    args += consts
    od = 2 * HD if mode == 'main' else HD
    return pl.pallas_call(
        functools.partial(_attn_body, mode),
        grid=(B,),
        in_specs=specs,
        out_specs=pl.BlockSpec((1, H, S, od), lambda b: (b, 0, 0, 0)),
        out_shape=jax.ShapeDtypeStruct((B, H, S, od), jnp.float32),
    )(*args)


def _proj_body(mode, *refs):
    if mode == 'dec':
        a_ref, i1_ref, i2_ref, wo_ref, o_ref = refs
        ident = i1_ref[0] + i2_ref[0]
    elif mode == 'main':
        a_ref, i1_ref, rm_ref, wo_ref, o_ref = refs
        ident = i1_ref[0] * rm_ref[0]
    else:
        a_ref, i1_ref, wo_ref, o_ref = refs
        ident = i1_ref[0]
    o_ref[0] = _dot(a_ref[0], wo_ref[...]) + ident


def _proj(mode, a, idents, woT):
    xspec = pl.BlockSpec((1, S, D), lambda b: (b, 0, 0))
    rmspec = pl.BlockSpec((1, S, 1), lambda b: (b, 0, 0))
    if mode == 'dec':
        specs = [xspec, xspec, xspec]
    elif mode == 'main':
        specs = [xspec, xspec, rmspec]
    else:
        specs = [xspec, xspec]
    args = [a] + list(idents)
    specs.append(pl.BlockSpec(woT.shape, lambda b: (0, 0)))
    args.append(woT)
    return pl.pallas_call(
        functools.partial(_proj_body, mode),
        grid=(B,),
        in_specs=specs,
        out_specs=xspec,
        out_shape=jax.ShapeDtypeStruct((B, S, D), jnp.float32),
    )(*args)


def _moe_body(y_ref, w1_ref, w3_ref, w2_ref, g_ref, o_ref):
    y = y_ref[0]
    yn = _rms(y)
    gl = _dot(yn, g_ref[...])
    m = jnp.max(gl, axis=1, keepdims=True)
    eg = jnp.exp(gl - m)
    gw = eg / jnp.sum(eg, axis=1, keepdims=True)
    cols = [gw[:, i:i + 1] for i in range(NE)]
    sel = []
    for i in range(NE):
        cnt = None
        for j in range(NE):
            if j == i:
                continue
            c = (cols[j] > cols[i]).astype(jnp.float32)
            cnt = c if cnt is None else cnt + c
        sel.append(cols[i] * (cnt < 2.0).astype(jnp.float32))
    wsum = sel[0] + sel[1] + sel[2] + sel[3]
    out = y
    for i in range(NE + 1):
        a1 = _dot(yn, w1_ref[i])
        a3 = _dot(yn, w3_ref[i])
        gel = 0.5 * a1 * (1.0 + lax.erf(a1 * 0.7071067811865476))
        eo = _dot(gel * a3, w2_ref[i])
        if i < NE:
            out = out + eo * (sel[i] / wsum)
        else:
            out = out + eo
    o_ref[0] = out


def _moe(y, w1T, w3T, w2T, gT):
    full = lambda a: pl.BlockSpec(a.shape, lambda b: (0,) * a.ndim)
    xspec = pl.BlockSpec((1, S, D), lambda b: (b, 0, 0))
    return pl.pallas_call(
        _moe_body,
        grid=(B,),
        in_specs=[xspec, full(w1T), full(w3T), full(w2T), full(gT)],
        out_specs=xspec,
        out_shape=jax.ShapeDtypeStruct((B, S, D), jnp.float32),
    )(y, w1T, w3T, w2T, gT)


def _routing_body(m_ref, ltge_ref, ieq_ref,
                  i1_ref, i2_ref, i3_ref, rm_ref):
    ones_row = jnp.ones((1, S), jnp.float32)
    iota_lane = lax.broadcasted_iota(jnp.int32, (S, S), 1).astype(jnp.float32)
    iota1s = lax.broadcasted_iota(jnp.int32, (1, S), 1).astype(jnp.float32)
    counts = []
    for b in range(B):
        maskcol = m_ref[b]
        cum_col = jnp.dot(ltge_ref[...], maskcol,
                          preferred_element_type=jnp.float32)
        cnt = jnp.max(cum_col)
        counts.append(cnt)
        cum_row = jnp.dot(ones_row, ieq_ref[...] * cum_col,
                          preferred_element_type=jnp.float32,
                          precision=lax.Precision.HIGHEST)
        i3_ref[b] = (cum_row.astype(jnp.int32) - 1 + b * S)
        cmat = (cum_col <= iota_lane).astype(jnp.float32)
        pos_row = jnp.minimum(
            jnp.dot(ones_row, cmat, preferred_element_type=jnp.float32),
            float(S - 1))
        i1_ref[b] = (pos_row.astype(jnp.int32) + b * S)
        rm_ref[b] = (iota1s < cnt).astype(jnp.float32)
    seq = counts[0]
    for b in range(1, B):
        seq = jnp.maximum(seq, counts[b])
    seq_i = seq.astype(jnp.int32)
    g = lax.broadcasted_iota(jnp.int32, (1, 4 * S), 1)
    hx = ((g >= seq_i).astype(jnp.int32)
          + (g >= 2 * seq_i).astype(jnp.int32)
          + (g >= 3 * seq_i).astype(jnp.int32))
    rem = jnp.minimum(g - hx * seq_i, S - 1)
    loc = hx * S + rem
    for b in range(B):
        i2_ref[b] = loc + b * 4 * S


def _routing(mask3):
    full = lambda a: pl.BlockSpec(a.shape, lambda: (0,) * a.ndim)
    ltge = jnp.asarray(_LTGE)
    ieq = jnp.asarray(_IEQ)
    return pl.pallas_call(
        _routing_body,
        in_specs=[full(mask3), full(ltge), full(ieq)],
        out_specs=(
            pl.BlockSpec((B, 1, S), lambda: (0, 0, 0)),
            pl.BlockSpec((B, 1, 4 * S), lambda: (0, 0, 0)),
            pl.BlockSpec((B, 1, S), lambda: (0, 0, 0)),
            pl.BlockSpec((B, 1, S), lambda: (0, 0, 0)),
        ),
        out_shape=(
            jax.ShapeDtypeStruct((B, 1, S), jnp.int32),
            jax.ShapeDtypeStruct((B, 1, 4 * S), jnp.int32),
            jax.ShapeDtypeStruct((B, 1, S), jnp.int32),
            jax.ShapeDtypeStruct((B, 1, S), jnp.float32),
        ),
    )(mask3, ltge, ieq)


def _sc_gather(src, idx, chunk):
    M = idx.shape[0]
    Dm = src.shape[1]
    per_w = M // SC_NW
    rounds = per_w // chunk
    assert per_w % chunk == 0 and M % SC_NW == 0
    mesh = plsc.VectorSubcoreMesh(core_axis_name="c", subcore_axis_name="s")

    @functools.partial(
        pl.kernel, mesh=mesh,
        out_type=jax.ShapeDtypeStruct((M, Dm), jnp.float32),
        scratch_types=[
            pltpu.VMEM((chunk,), jnp.int32),
            pltpu.VMEM((chunk,), jnp.int32),
            pltpu.VMEM((chunk, Dm), jnp.float32),
            pltpu.VMEM((chunk, Dm), jnp.float32),
            pltpu.SemaphoreType.DMA,
            pltpu.SemaphoreType.DMA,
        ],
    )
    def k(src_hbm, idx_hbm, out_hbm, idx0, idx1, rows0, rows1, sem0, sem1):
        wid = lax.axis_index("s") * SC_NC + lax.axis_index("c")
        base = wid * per_w
        idx_v = [idx0, idx1]
        rows_v = [rows0, rows1]
        sems = [sem0, sem1]
        pltpu.sync_copy(idx_hbm.at[pl.ds(base, chunk)], idx0)
        cp = pltpu.async_copy(src_hbm.at[idx0], rows0, sem0)
        copies = [cp, None]
        for r in range(rounds):
            cur = r % 2
            nxt = (r + 1) % 2
            if r + 1 < rounds:
                pltpu.sync_copy(
                    idx_hbm.at[pl.ds(base + (r + 1) * chunk, chunk)],
                    idx_v[nxt])
                copies[nxt] = pltpu.async_copy(
                    src_hbm.at[idx_v[nxt]], rows_v[nxt], sems[nxt])
            copies[cur].wait()
            pltpu.sync_copy(rows_v[cur],
                            out_hbm.at[pl.ds(base + r * chunk, chunk)])

    return k(src, idx)


def _r_rmsnorm(x, eps=1e-06):
    return x / jnp.sqrt(jnp.mean(x * x, axis=-1, keepdims=True) + eps)


def _r_freqs(seq_len, head_dim):
    theta = 1.0 / (10000.0 ** (np.arange(0, head_dim, 2).astype(np.float32)
                               / head_dim))
    pos = np.arange(seq_len).astype(np.float32)
    freqs = np.outer(pos, theta)
    return jnp.asarray(np.exp(1j * freqs).astype(np.complex64))


def _r_rotary(x, freqs_cis):
    Bb, Ss, Hh, hd = x.shape
    xc = x.reshape(Bb, Ss, Hh, hd // 2, 2)
    xcplx = jax.lax.complex(xc[..., 0], xc[..., 1])
    f = freqs_cis[:Ss].reshape(1, Ss, 1, hd // 2)
    rot = xcplx * f
    return jnp.stack([jnp.real(rot), jnp.imag(rot)], axis=-1).reshape(
        Bb, Ss, Hh, hd)


def _r_mla(x, p, freqs_cis):
    Bb, Ss, Dd = x.shape
    identity = x
    xn = _r_rmsnorm(x)
    q = (xn @ p['wq'].T).reshape(Bb, Ss, H, HD)
    kv = (xn @ p['wkv'].T).reshape(Bb, Ss, H, 2 * HD)
    k = kv[..., :HD]
    v = kv[..., HD:]
    q = _r_rotary(q, freqs_cis)
    attn = jnp.matmul(q, jnp.swapaxes(k, -1, -2)) / np.sqrt(HD)
    attn = jax.nn.softmax(attn.astype(jnp.float32), axis=-1).astype(x.dtype)
    ao = jnp.matmul(attn, v)
    ao = jnp.transpose(ao, (0, 2, 1, 3)).reshape(Bb, Ss, Dd)
    return ao @ p['wo'].T + identity


def _r_expert_ff(x, ep):
    return (jax.nn.gelu(x @ ep['w1'].T, approximate=False)
            * (x @ ep['w3'].T)) @ ep['w2'].T


def _r_moe(x, p):
    identity = x
    xn = _r_rmsnorm(x)
    gw = jax.nn.softmax(xn @ p['gate'].T, axis=-1)
    topw, topi = jax.lax.top_k(gw, 2)
    topw = topw / jnp.sum(topw, axis=-1, keepdims=True)
    out = jnp.zeros_like(x)
    for i in range(NE):
        eo = _r_expert_ff(xn, p['experts'][i])
        wi = jnp.sum(topw * (topi == i).astype(x.dtype), axis=-1,
                     keepdims=True)
        out = out + eo * wi
    out = out + _r_expert_ff(xn, p['shared'])
    return out + identity


def _r_mask(x, params):
    freqs = _r_freqs(2048, HD)
    hh = _r_moe(_r_mla(x, params['encoder'], freqs), params['encoder'])
    pr = params['routing']
    q = hh[:, :-1] @ pr['wq'].T
    k = hh[:, 1:] @ pr['wk'].T
    eps = 1e-08
    qn = jnp.maximum(jnp.linalg.norm(q, axis=-1), eps)
    kn = jnp.maximum(jnp.linalg.norm(k, axis=-1), eps)
    cos = jnp.sum(q * k, axis=-1) / (qn * kn)
    p = jnp.clip((1.0 - cos) / 2.0, 0.0, 1.0)
    p = jnp.pad(p, ((0, 0), (1, 0)), constant_values=1.0)
    return (p > 0.5).astype(jnp.float32).reshape(B, S, 1)


def _block_std(x, p, add=None):
    wqT = p['wq'].T
    wkvT = p['wkv'].T
    woT = p['wo'].T
    w1T = jnp.stack([e['w1'].T for e in (p['experts'] + [p['shared']])])
    w3T = jnp.stack([e['w3'].T for e in (p['experts'] + [p['shared']])])
    w2T = jnp.stack([e['w2'].T for e in (p['experts'] + [p['shared']])])
    gT = p['gate'].T
    if add is None:
        ao = _attn('enc', x, None, wqT, wkvT)
        y = _proj('enc', ao.reshape(B, S, D), (x,), woT)
    else:
        ao = _attn('dec', x, add, wqT, wkvT)
        y = _proj('dec', ao.reshape(B, S, D), (x, add), woT)
    return _moe(y, w1T, w3T, w2T, gT)


def kernel(x, params):
    pe, pm, pd = params['encoder'], params['main'], params['decoder']
    pr = params['routing']

    h = _block_std(x, pe)
    mask3 = _r_mask(x, params)

    idx1, idx2, idx3, rm = _routing(mask3)
    idx1 = idx1.reshape(B * S)
    idx2 = idx2.reshape(B * 4 * S)
    idx3 = idx3.reshape(B * S)
    rm = rm.reshape(B, S, 1)

    comp = _sc_gather(h.reshape(B * S, D), idx1, 128).reshape(B, S, D)

    ao = _attn('main', comp, rm, pm['wq'].T, pm['wkv'].T)
    aop = _sc_gather(ao.reshape(B * H * S, 2 * HD), idx2, 128)
    aop = aop[:, :HD].reshape(B, S, D)
    y2 = _proj('main', aop, (comp, rm), pm['wo'].T)
    w1T = jnp.stack([e['w1'].T for e in (pm['experts'] + [pm['shared']])])
    w3T = jnp.stack([e['w3'].T for e in (pm['experts'] + [pm['shared']])])
    w2T = jnp.stack([e['w2'].T for e in (pm['experts'] + [pm['shared']])])
    c2 = _moe(y2, w1T, w3T, w2T, pm['gate'].T)

    h2 = _sc_gather(c2.reshape(B * S, D), idx3, 128).reshape(B, S, D)

    return _block_std(h, pd, add=h2)
